# baseline (speedup 1.0000x reference)
import jax
import jax.numpy as jnp
from jax import lax
from jax.experimental import pallas as pl
from jax.experimental.pallas import tpu as pltpu
from jax.experimental.pallas import tpu_sc as plsc

N = 10000
E = 320000
H = 256
Q = 64
NQ = H // Q
OUT = 64
ALPHA = 0.5

NC = 2
NS = 16
ET = E // NS
K = 80
C = ET // K
STRIPE = 624
REM = N - NS * STRIPE

_MESH = plsc.VectorSubcoreMesh(core_axis_name="c", subcore_axis_name="s")


def _stripe_copy(src, dst, s, src_off, dst_off):
    base = s * STRIPE
    pltpu.sync_copy(src.at[pl.ds(src_off + base, STRIPE)],
                    dst.at[pl.ds(dst_off + base, STRIPE)])

    @pl.when(s == NS - 1)
    def _():
        tail = NS * STRIPE
        pltpu.sync_copy(src.at[pl.ds(src_off + tail, REM)],
                        dst.at[pl.ds(dst_off + tail, REM)])


def _add_offset(dst_v, src_v, off):
    @pl.loop(0, C)
    def _row(r):
        for j in range(K // 16):
            sl = pl.ds(j * 16, 16)
            dst_v[r, sl] = src_v[r, sl] + off


NBUF = 5


def _sc_pass1_body(table_ref, ve_ref, zeros_ref, out_ref,
                   gidx_v, sidx_v, acc, *ring):
    c = lax.axis_index("c")
    s = lax.axis_index("s")
    rows = ring[:NBUF]
    gsem = ring[NBUF:2 * NBUF]
    ssem = ring[2 * NBUF:]
    pltpu.sync_copy(ve_ref.at[s], gidx_v)
    pltpu.sync_copy(ve_ref.at[NS + s], sidx_v)
    for sub in range(2):
        q = 2 * c + sub
        t_q = table_ref.at[pl.ds(q * N, N)]
        _stripe_copy(zeros_ref, acc, s, 0, 0)
        pltpu.async_copy(t_q.at[gidx_v.at[0]], rows[0], gsem[0])
        pltpu.async_copy(t_q.at[gidx_v.at[1]], rows[1], gsem[1])
        plsc.subcore_barrier()

        @pl.loop(0, C, step=NBUF)
        def _chunk(g):
            for b in range(NBUF):
                i = g + b
                sp = (b + 2) % NBUF

                @pl.when(i >= 3)
                def _():
                    pltpu.make_async_copy(
                        rows[sp], acc.at[sidx_v.at[i - 3]], ssem[sp]).wait()

                @pl.when(i + 2 < C)
                def _():
                    pltpu.async_copy(t_q.at[gidx_v.at[i + 2]],
                                     rows[sp], gsem[sp])

                pltpu.make_async_copy(t_q.at[gidx_v.at[i]],
                                      rows[b], gsem[b]).wait()
                pltpu.async_copy(rows[b], acc.at[sidx_v.at[i]],
                                 ssem[b], add=True)

        for i in range(C - 3, C):
            b = i % NBUF
            pltpu.make_async_copy(rows[b], acc.at[sidx_v.at[i]],
                                  ssem[b]).wait()
        plsc.subcore_barrier()
        _stripe_copy(acc, out_ref, s, 0, q * N)


_sc_pass1 = pl.kernel(
    _sc_pass1_body,
    out_type=jax.ShapeDtypeStruct((NQ * N, Q), jnp.float32),
    mesh=_MESH,
    compiler_params=pltpu.CompilerParams(use_tc_tiling_on_sc=False),
    scratch_types=[
        pltpu.VMEM((C, K), jnp.int32),
        pltpu.VMEM((C, K), jnp.int32),
        pltpu.VMEM_SHARED((N, Q), jnp.float32),
    ] + [pltpu.VMEM((K, Q), jnp.float32)] * NBUF
      + [pltpu.SemaphoreType.DMA] * (2 * NBUF),
)


def _sc_pass2_body(tab_ref, ve_ref, zeros_ref,
                   out_ref, vidx_v, eidx_v, acc, *ring):
    c = lax.axis_index("c")
    s = lax.axis_index("s")
    rows = ring[:NBUF]
    gsem = ring[NBUF:2 * NBUF]
    ssem = ring[2 * NBUF:]
    pltpu.sync_copy(ve_ref.at[s], vidx_v)
    pltpu.sync_copy(ve_ref.at[NS + s], eidx_v)
    for sub in range(2):
        q = 2 * c + sub
        ta_q = tab_ref.at[pl.ds(q * N, N)]
        tb_q = tab_ref.at[pl.ds((NQ + q) * N, N)]
        _stripe_copy(zeros_ref, acc, s, 0, 0)
        pltpu.async_copy(ta_q.at[vidx_v.at[0]], rows[0], gsem[0])
        pltpu.async_copy(ta_q.at[vidx_v.at[1]], rows[1], gsem[1])
        pltpu.make_async_copy(ta_q.at[vidx_v.at[0]], rows[0], gsem[0]).wait()
        pltpu.async_copy(tb_q.at[eidx_v.at[0]], rows[0], gsem[0], add=True)
        plsc.subcore_barrier()

        @pl.loop(0, C, step=NBUF)
        def _chunk(g):
            for b in range(NBUF):
                i = g + b
                sp2 = (b + 2) % NBUF
                sp1 = (b + 1) % NBUF

                @pl.when(i >= 3)
                def _():
                    pltpu.make_async_copy(
                        rows[sp2], acc.at[vidx_v.at[i - 3]],
                        ssem[sp2]).wait()

                @pl.when(i + 2 < C)
                def _():
                    pltpu.async_copy(ta_q.at[vidx_v.at[i + 2]],
                                     rows[sp2], gsem[sp2])

                @pl.when(i + 1 < C)
                def _():
                    pltpu.make_async_copy(ta_q.at[vidx_v.at[i + 1]],
                                          rows[sp1], gsem[sp1]).wait()
                    pltpu.async_copy(tb_q.at[eidx_v.at[i + 1]],
                                     rows[sp1], gsem[sp1], add=True)

                pltpu.make_async_copy(tb_q.at[eidx_v.at[i]],
                                      rows[b], gsem[b]).wait()

                @pl.loop(0, K, step=8)
                def _relu_row(r):
                    for rr in range(8):
                        for j in range(Q // 16):
                            sl = pl.ds(j * 16, 16)
                            rows[b][r + rr, sl] = jnp.maximum(
                                rows[b][r + rr, sl], 0.0)

                pltpu.async_copy(rows[b], acc.at[vidx_v.at[i]],
                                 ssem[b], add=True)

        for i in range(C - 3, C):
            b = i % NBUF
            pltpu.make_async_copy(rows[b], acc.at[vidx_v.at[i]],
                                  ssem[b]).wait()
        plsc.subcore_barrier()
        _stripe_copy(acc, out_ref, s, 0, q * N)


_sc_pass2 = pl.kernel(
    _sc_pass2_body,
    out_type=jax.ShapeDtypeStruct((NQ * N, Q), jnp.float32),
    mesh=_MESH,
    compiler_params=pltpu.CompilerParams(use_tc_tiling_on_sc=False),
    scratch_types=[
        pltpu.VMEM((C, K), jnp.int32),
        pltpu.VMEM((C, K), jnp.int32),
        pltpu.VMEM_SHARED((N, Q), jnp.float32),
    ] + [pltpu.VMEM((K, Q), jnp.float32)] * NBUF
      + [pltpu.SemaphoreType.DMA] * (2 * NBUF),
)


R = 2000
G = N // R

def _dot(a, b):
    return jnp.dot(a, b, preferred_element_type=jnp.float32)

def _full(shape):
    return pl.BlockSpec(shape, lambda i: (0,) * len(shape))

def _rows(d):
    return pl.BlockSpec((R, d), lambda i: (i, 0))

def _quarters():
    return pl.BlockSpec((NQ, R, Q), lambda i: (0, i, 0))

def _split4(m):
    return jnp.stack([m[:, q * Q:(q + 1) * Q] for q in range(NQ)])

def _cat4(ref):
    return jnp.concatenate([ref[q] for q in range(NQ)], axis=1)


def _tc_pre_body(x_ref, linW, linb, W1a, b1a, W1b, b1b, W2at, b2a, W2ab,
                 h_ref, PB4_ref, A4_ref):
    h = jnp.maximum(_dot(x_ref[...], linW[...]) + linb[...], 0.0)
    h_ref[...] = h
    Pm = _dot(jnp.maximum(_dot(h, W1a[...]) + b1a[...], 0.0), W1b[...]) + b1b[...]
    PB4_ref[...] = _split4(_dot(Pm, W2ab[...]))
    Am = _dot(h, W2at[...]) + b2a[...]
    A4_ref[...] = _split4(Am)


def _tc_pre(x, linW, linb, W1a, b1a, W1b, b1b, W2at, b2a, W2ab):
    return pl.pallas_call(
        _tc_pre_body,
        grid=(G,),
        in_specs=[_rows(128), _full((128, H)), _full((1, H)),
                  _full((H, H)), _full((1, H)), _full((H, H)), _full((1, H)),
                  _full((H, H)), _full((1, H)), _full((H, H))],
        out_specs=[_rows(H), _quarters(), _quarters()],
        out_shape=[jax.ShapeDtypeStruct((N, H), jnp.float32),
                   jax.ShapeDtypeStruct((NQ, N, Q), jnp.float32),
                   jax.ShapeDtypeStruct((NQ, N, Q), jnp.float32)],
    )(x, linW, linb, W1a, b1a, W1b, b1b, W2at, b2a, W2ab)


def _tc_b_body(xe4_ref, W2ab, B4_ref):
    Bm = _dot(_cat4(xe4_ref), W2ab[...])
    B4_ref[...] = _split4(Bm)


def _tc_b(xe4, W2ab):
    return pl.pallas_call(
        _tc_b_body,
        grid=(G,),
        in_specs=[_quarters(), _full((H, H))],
        out_specs=_quarters(),
        out_shape=jax.ShapeDtypeStruct((NQ, N, Q), jnp.float32),
    )(xe4, W2ab)


def _layer_tail(S4_ref, x0_ref, W2b, b2b, W3a, b3a, W3b, b3b):
    Xv = _dot(_cat4(S4_ref), W2b[...]) + b2b[...]
    hm = (1.0 - ALPHA) * Xv + ALPHA * x0_ref[...]
    t = _dot(jnp.maximum(_dot(hm, W3a[...]) + b3a[...], 0.0), W3b[...]) + b3b[...]
    return jnp.maximum(t, 0.0)


def _tc_mid_body(S4_ref, x0_ref, W2b, b2b, W3a, b3a, W3b, b3b,
                 W1a, b1a, W1b, b1b, W2at, b2a, P4_ref, A4_ref):
    h = _layer_tail(S4_ref, x0_ref, W2b, b2b, W3a, b3a, W3b, b3b)
    Pm = _dot(jnp.maximum(_dot(h, W1a[...]) + b1a[...], 0.0), W1b[...]) + b1b[...]
    P4_ref[...] = _split4(Pm)
    Am = _dot(h, W2at[...]) + b2a[...]
    A4_ref[...] = _split4(Am)


def _tc_mid(S4, x0, *ws):
    return pl.pallas_call(
        _tc_mid_body,
        grid=(G,),
        in_specs=[_quarters(), _rows(H)] + [_full((H, H)), _full((1, H))] * 6,
        out_specs=[_quarters(), _quarters()],
        out_shape=[jax.ShapeDtypeStruct((NQ, N, Q), jnp.float32),
                   jax.ShapeDtypeStruct((NQ, N, Q), jnp.float32)],
    )(S4, x0, *ws)


def _tc_final_body(S4_ref, x0_ref, W2b, b2b, W3a, b3a, W3b, b3b,
                   Da, Dab, Db, Dbb, out_ref):
    h = _layer_tail(S4_ref, x0_ref, W2b, b2b, W3a, b3a, W3b, b3b)
    out_ref[...] = _dot(jnp.maximum(_dot(h, Da[...]) + Dab[...], 0.0),
                        Db[...]) + Dbb[...]


def _tc_final(S4, x0, W2b, b2b, W3a, b3a, W3b, b3b, Da, Dab, Db, Dbb):
    return pl.pallas_call(
        _tc_final_body,
        grid=(G,),
        in_specs=[_quarters(), _rows(H)] + [_full((H, H)), _full((1, H))] * 4
                 + [_full((H, OUT)), _full((1, OUT))],
        out_specs=_rows(OUT),
        out_shape=jax.ShapeDtypeStruct((N, OUT), jnp.float32),
    )(S4, x0, W2b, b2b, W3a, b3a, W3b, b3b, Da, Dab, Db, Dbb)


@jax.jit
def kernel(x, hyperedge_index, lin_W, lin_b, W1a_W, W1a_b, W1b_W, W1b_b,
           W2a_W, W2a_b, W2b_W, W2b_b, W3a_W, W3a_b, W3b_W, W3b_b,
           Da_W, Da_b, Db_W, Db_b):
    vertex = hyperedge_index[0]
    edges = hyperedge_index[1]

    ve = hyperedge_index.reshape(2 * NS, C, K)
    zeros = jnp.zeros((N, Q), jnp.float32)

    b = lambda v: v.reshape(1, -1)
    W2at = W2a_W[:H]
    W2ab = W2a_W[H:]

    h0, PB4, A4 = _tc_pre(x, lin_W, b(lin_b), W1a_W, b(W1a_b),
                          W1b_W, b(W1b_b), W2at, b(W2a_b), W2ab)

    B4 = _sc_pass1(PB4.reshape(NQ * N, Q), ve, zeros)
    S4 = _sc_pass2(
        jnp.concatenate([A4.reshape(NQ * N, Q), B4], axis=0), ve, zeros)
    P4, A4 = _tc_mid(S4.reshape(NQ, N, Q), h0, W2b_W, b(W2b_b),
                     W3a_W, b(W3a_b), W3b_W, b(W3b_b),
                     W1a_W, b(W1a_b), W1b_W, b(W1b_b), W2at, b(W2a_b))

    Xe4 = _sc_pass1(P4.reshape(NQ * N, Q), ve, zeros)
    B4 = _tc_b(Xe4.reshape(NQ, N, Q), W2ab)
    S4 = _sc_pass2(
        jnp.concatenate([A4.reshape(NQ * N, Q), B4.reshape(NQ * N, Q)],
                        axis=0), ve, zeros)
    out = _tc_final(S4.reshape(NQ, N, Q), h0, W2b_W, b(W2b_b),
                    W3a_W, b(W3a_b), W3b_W, b(W3b_b),
                    Da_W, b(Da_b), Db_W, b(Db_b))

    Xe = jnp.concatenate(
        [Xe4[q * N:(q + 1) * N] for q in range(NQ)], axis=1)
    return out, Xe


# speedup vs baseline: 1.0737x; 1.0737x over previous
import jax
import jax.numpy as jnp
from jax import lax
from jax.experimental import pallas as pl
from jax.experimental.pallas import tpu as pltpu
from jax.experimental.pallas import tpu_sc as plsc

N = 10000
E = 320000
H = 256
Q = 64
NQ = H // Q
OUT = 64
ALPHA = 0.5

NC = 2
NS = 16
ET = E // NS
K = 80
C = ET // K
STRIPE = 624
REM = N - NS * STRIPE

_MESH = plsc.VectorSubcoreMesh(core_axis_name="c", subcore_axis_name="s")


def _stripe_copy(src, dst, s, src_off, dst_off):
    base = s * STRIPE
    pltpu.sync_copy(src.at[pl.ds(src_off + base, STRIPE)],
                    dst.at[pl.ds(dst_off + base, STRIPE)])

    @pl.when(s == NS - 1)
    def _():
        tail = NS * STRIPE
        pltpu.sync_copy(src.at[pl.ds(src_off + tail, REM)],
                        dst.at[pl.ds(dst_off + tail, REM)])


def _add_offset(dst_v, src_v, off):
    @pl.loop(0, C)
    def _row(r):
        for j in range(K // 16):
            sl = pl.ds(j * 16, 16)
            dst_v[r, sl] = src_v[r, sl] + off


NBUF = 5


def _sc_pass1_body(table_ref, ve_ref, zeros_ref, out_ref,
                   gidx_v, sidx_v, acc, *ring):
    c = lax.axis_index("c")
    s = lax.axis_index("s")
    rows = ring[:NBUF]
    gsem = ring[NBUF:2 * NBUF]
    ssem = ring[2 * NBUF:]
    pltpu.sync_copy(ve_ref.at[s], gidx_v)
    pltpu.sync_copy(ve_ref.at[NS + s], sidx_v)
    for sub in range(2):
        q = 2 * c + sub
        t_q = table_ref.at[pl.ds(q * N, N)]
        _stripe_copy(zeros_ref, acc, s, 0, 0)
        pltpu.async_copy(t_q.at[gidx_v.at[0]], rows[0], gsem[0])
        pltpu.async_copy(t_q.at[gidx_v.at[1]], rows[1], gsem[1])
        plsc.subcore_barrier()

        @pl.loop(0, C, step=NBUF)
        def _chunk(g):
            for b in range(NBUF):
                i = g + b
                sp = (b + 2) % NBUF

                @pl.when(i >= 3)
                def _():
                    pltpu.make_async_copy(
                        rows[sp], acc.at[sidx_v.at[i - 3]], ssem[sp]).wait()

                @pl.when(i + 2 < C)
                def _():
                    pltpu.async_copy(t_q.at[gidx_v.at[i + 2]],
                                     rows[sp], gsem[sp])

                pltpu.make_async_copy(t_q.at[gidx_v.at[i]],
                                      rows[b], gsem[b]).wait()
                pltpu.async_copy(rows[b], acc.at[sidx_v.at[i]],
                                 ssem[b], add=True)

        for i in range(C - 3, C):
            b = i % NBUF
            pltpu.make_async_copy(rows[b], acc.at[sidx_v.at[i]],
                                  ssem[b]).wait()
        plsc.subcore_barrier()
        _stripe_copy(acc, out_ref, s, 0, q * N)


_sc_pass1 = pl.kernel(
    _sc_pass1_body,
    out_type=jax.ShapeDtypeStruct((NQ * N, Q), jnp.float32),
    mesh=_MESH,
    compiler_params=pltpu.CompilerParams(use_tc_tiling_on_sc=False),
    scratch_types=[
        pltpu.VMEM((C, K), jnp.int32),
        pltpu.VMEM((C, K), jnp.int32),
        pltpu.VMEM_SHARED((N, Q), jnp.float32),
    ] + [pltpu.VMEM((K, Q), jnp.float32)] * NBUF
      + [pltpu.SemaphoreType.DMA] * (2 * NBUF),
)


def _sc_pass2_body(ta_ref, tb_ref, ve_ref, zeros_ref,
                   out_ref, vidx_v, eidx_v, acc, *ring):
    c = lax.axis_index("c")
    s = lax.axis_index("s")
    rows = ring[:NBUF]
    gsem = ring[NBUF:2 * NBUF]
    ssem = ring[2 * NBUF:]
    pltpu.sync_copy(ve_ref.at[s], vidx_v)
    pltpu.sync_copy(ve_ref.at[NS + s], eidx_v)
    for sub in range(2):
        q = 2 * c + sub
        ta_q = ta_ref.at[pl.ds(q * N, N)]
        tb_q = tb_ref.at[pl.ds(q * N, N)]
        _stripe_copy(zeros_ref, acc, s, 0, 0)
        pltpu.async_copy(ta_q.at[vidx_v.at[0]], rows[0], gsem[0])
        pltpu.async_copy(ta_q.at[vidx_v.at[1]], rows[1], gsem[1])
        pltpu.make_async_copy(ta_q.at[vidx_v.at[0]], rows[0], gsem[0]).wait()
        pltpu.async_copy(tb_q.at[eidx_v.at[0]], rows[0], gsem[0], add=True)
        plsc.subcore_barrier()

        @pl.loop(0, C, step=NBUF)
        def _chunk(g):
            for b in range(NBUF):
                i = g + b
                sp2 = (b + 2) % NBUF
                sp1 = (b + 1) % NBUF

                @pl.when(i >= 3)
                def _():
                    pltpu.make_async_copy(
                        rows[sp2], acc.at[vidx_v.at[i - 3]],
                        ssem[sp2]).wait()

                @pl.when(i + 2 < C)
                def _():
                    pltpu.async_copy(ta_q.at[vidx_v.at[i + 2]],
                                     rows[sp2], gsem[sp2])

                @pl.when(i + 1 < C)
                def _():
                    pltpu.make_async_copy(ta_q.at[vidx_v.at[i + 1]],
                                          rows[sp1], gsem[sp1]).wait()
                    pltpu.async_copy(tb_q.at[eidx_v.at[i + 1]],
                                     rows[sp1], gsem[sp1], add=True)

                pltpu.make_async_copy(tb_q.at[eidx_v.at[i]],
                                      rows[b], gsem[b]).wait()

                @pl.loop(0, K, step=8)
                def _relu_row(r):
                    for rr in range(8):
                        for j in range(Q // 16):
                            sl = pl.ds(j * 16, 16)
                            rows[b][r + rr, sl] = jnp.maximum(
                                rows[b][r + rr, sl], 0.0)

                pltpu.async_copy(rows[b], acc.at[vidx_v.at[i]],
                                 ssem[b], add=True)

        for i in range(C - 3, C):
            b = i % NBUF
            pltpu.make_async_copy(rows[b], acc.at[vidx_v.at[i]],
                                  ssem[b]).wait()
        plsc.subcore_barrier()
        _stripe_copy(acc, out_ref, s, 0, q * N)


_sc_pass2 = pl.kernel(
    _sc_pass2_body,
    out_type=jax.ShapeDtypeStruct((NQ * N, Q), jnp.float32),
    mesh=_MESH,
    compiler_params=pltpu.CompilerParams(use_tc_tiling_on_sc=False),
    scratch_types=[
        pltpu.VMEM((C, K), jnp.int32),
        pltpu.VMEM((C, K), jnp.int32),
        pltpu.VMEM_SHARED((N, Q), jnp.float32),
    ] + [pltpu.VMEM((K, Q), jnp.float32)] * NBUF
      + [pltpu.SemaphoreType.DMA] * (2 * NBUF),
)


R = 2000
G = N // R

def _dot(a, b):
    return jnp.dot(a, b, preferred_element_type=jnp.float32)

def _full(shape):
    return pl.BlockSpec(shape, lambda i: (0,) * len(shape))

def _rows(d):
    return pl.BlockSpec((R, d), lambda i: (i, 0))

def _quarters():
    return pl.BlockSpec((NQ, R, Q), lambda i: (0, i, 0))

def _split4(m):
    return jnp.stack([m[:, q * Q:(q + 1) * Q] for q in range(NQ)])

def _cat4(ref):
    return jnp.concatenate([ref[q] for q in range(NQ)], axis=1)


def _tc_pre_body(x_ref, linW, linb, W1a, b1a, W1b, b1b, W2at, b2a, W2ab,
                 h_ref, PB4_ref, A4_ref):
    h = jnp.maximum(_dot(x_ref[...], linW[...]) + linb[...], 0.0)
    h_ref[...] = h
    Pm = _dot(jnp.maximum(_dot(h, W1a[...]) + b1a[...], 0.0), W1b[...]) + b1b[...]
    PB4_ref[...] = _split4(_dot(Pm, W2ab[...]))
    Am = _dot(h, W2at[...]) + b2a[...]
    A4_ref[...] = _split4(Am)


def _tc_pre(x, linW, linb, W1a, b1a, W1b, b1b, W2at, b2a, W2ab):
    return pl.pallas_call(
        _tc_pre_body,
        grid=(G,),
        in_specs=[_rows(128), _full((128, H)), _full((1, H)),
                  _full((H, H)), _full((1, H)), _full((H, H)), _full((1, H)),
                  _full((H, H)), _full((1, H)), _full((H, H))],
        out_specs=[_rows(H), _quarters(), _quarters()],
        out_shape=[jax.ShapeDtypeStruct((N, H), jnp.float32),
                   jax.ShapeDtypeStruct((NQ, N, Q), jnp.float32),
                   jax.ShapeDtypeStruct((NQ, N, Q), jnp.float32)],
    )(x, linW, linb, W1a, b1a, W1b, b1b, W2at, b2a, W2ab)


def _tc_b_body(xe4_ref, W2ab, B4_ref):
    Bm = _dot(_cat4(xe4_ref), W2ab[...])
    B4_ref[...] = _split4(Bm)


def _tc_b(xe4, W2ab):
    return pl.pallas_call(
        _tc_b_body,
        grid=(G,),
        in_specs=[_quarters(), _full((H, H))],
        out_specs=_quarters(),
        out_shape=jax.ShapeDtypeStruct((NQ, N, Q), jnp.float32),
    )(xe4, W2ab)


def _layer_tail(S4_ref, x0_ref, W2b, b2b, W3a, b3a, W3b, b3b):
    Xv = _dot(_cat4(S4_ref), W2b[...]) + b2b[...]
    hm = (1.0 - ALPHA) * Xv + ALPHA * x0_ref[...]
    t = _dot(jnp.maximum(_dot(hm, W3a[...]) + b3a[...], 0.0), W3b[...]) + b3b[...]
    return jnp.maximum(t, 0.0)


def _tc_mid_body(S4_ref, x0_ref, W2b, b2b, W3a, b3a, W3b, b3b,
                 W1a, b1a, W1b, b1b, W2at, b2a, P4_ref, A4_ref):
    h = _layer_tail(S4_ref, x0_ref, W2b, b2b, W3a, b3a, W3b, b3b)
    Pm = _dot(jnp.maximum(_dot(h, W1a[...]) + b1a[...], 0.0), W1b[...]) + b1b[...]
    P4_ref[...] = _split4(Pm)
    Am = _dot(h, W2at[...]) + b2a[...]
    A4_ref[...] = _split4(Am)


def _tc_mid(S4, x0, *ws):
    return pl.pallas_call(
        _tc_mid_body,
        grid=(G,),
        in_specs=[_quarters(), _rows(H)] + [_full((H, H)), _full((1, H))] * 6,
        out_specs=[_quarters(), _quarters()],
        out_shape=[jax.ShapeDtypeStruct((NQ, N, Q), jnp.float32),
                   jax.ShapeDtypeStruct((NQ, N, Q), jnp.float32)],
    )(S4, x0, *ws)


def _tc_final_body(S4_ref, x0_ref, W2b, b2b, W3a, b3a, W3b, b3b,
                   Da, Dab, Db, Dbb, out_ref):
    h = _layer_tail(S4_ref, x0_ref, W2b, b2b, W3a, b3a, W3b, b3b)
    out_ref[...] = _dot(jnp.maximum(_dot(h, Da[...]) + Dab[...], 0.0),
                        Db[...]) + Dbb[...]


def _tc_final(S4, x0, W2b, b2b, W3a, b3a, W3b, b3b, Da, Dab, Db, Dbb):
    return pl.pallas_call(
        _tc_final_body,
        grid=(G,),
        in_specs=[_quarters(), _rows(H)] + [_full((H, H)), _full((1, H))] * 4
                 + [_full((H, OUT)), _full((1, OUT))],
        out_specs=_rows(OUT),
        out_shape=jax.ShapeDtypeStruct((N, OUT), jnp.float32),
    )(S4, x0, W2b, b2b, W3a, b3a, W3b, b3b, Da, Dab, Db, Dbb)


@jax.jit
def kernel(x, hyperedge_index, lin_W, lin_b, W1a_W, W1a_b, W1b_W, W1b_b,
           W2a_W, W2a_b, W2b_W, W2b_b, W3a_W, W3a_b, W3b_W, W3b_b,
           Da_W, Da_b, Db_W, Db_b):
    vertex = hyperedge_index[0]
    edges = hyperedge_index[1]

    ve = hyperedge_index.reshape(2 * NS, C, K)
    zeros = jnp.zeros((N, Q), jnp.float32)

    b = lambda v: v.reshape(1, -1)
    W2at = W2a_W[:H]
    W2ab = W2a_W[H:]

    h0, PB4, A4 = _tc_pre(x, lin_W, b(lin_b), W1a_W, b(W1a_b),
                          W1b_W, b(W1b_b), W2at, b(W2a_b), W2ab)

    B4 = _sc_pass1(PB4.reshape(NQ * N, Q), ve, zeros)
    S4 = _sc_pass2(A4.reshape(NQ * N, Q), B4, ve, zeros)
    P4, A4 = _tc_mid(S4.reshape(NQ, N, Q), h0, W2b_W, b(W2b_b),
                     W3a_W, b(W3a_b), W3b_W, b(W3b_b),
                     W1a_W, b(W1a_b), W1b_W, b(W1b_b), W2at, b(W2a_b))

    Xe4 = _sc_pass1(P4.reshape(NQ * N, Q), ve, zeros)
    B4 = _tc_b(Xe4.reshape(NQ, N, Q), W2ab)
    S4 = _sc_pass2(A4.reshape(NQ * N, Q), B4.reshape(NQ * N, Q),
                   ve, zeros)
    out = _tc_final(S4.reshape(NQ, N, Q), h0, W2b_W, b(W2b_b),
                    W3a_W, b(W3a_b), W3b_W, b(W3b_b),
                    Da_W, b(Da_b), Db_W, b(Db_b))

    Xe = jnp.concatenate(
        [Xe4[q * N:(q + 1) * N] for q in range(NQ)], axis=1)
    return out, Xe
